# initial kernel scaffold (unmeasured)
import jax
import jax.numpy as jnp
from jax import lax
from jax.experimental import pallas as pl
from jax.experimental.pallas import tpu as pltpu

N_DEV = 4
M = 8192
N = 4096
QR = M // N_DEV
SUB = 4
BR = QR // SUB
T_TOTAL = 2 * (N_DEV - 1) * SUB


def _all_reduce_relu(p):

    def body(p_ref, o_ref, acc, recvb, localb, outstage,
             send_sems, recv_sems, local_sem, out_sem, credit_sem):
        d = lax.axis_index("i")
        right = lax.rem(d + 1, N_DEV)
        left = lax.rem(d + N_DEV - 1, N_DEV)

        barrier_sem = pltpu.get_barrier_semaphore()
        for nbr in (left, right):
            pl.semaphore_signal(
                barrier_sem, inc=1,
                device_id=(nbr,), device_id_type=pl.DeviceIdType.MESH,
            )
        pl.semaphore_wait(barrier_sem, 2)

        q0 = lax.rem(d + N_DEV - 1, N_DEV)
        for j in range(SUB):
            cp = pltpu.make_async_copy(
                p_ref.at[pl.ds(q0 * QR + j * BR, BR), :], acc.at[j], local_sem
            )
            cp.start()
            cp.wait()

        t = 0
        for s in range(N_DEV - 1):
            q_recv = lax.rem(d + 2 * N_DEV - 2 - s, N_DEV)
            for j in range(SUB):
                slot = t % 2
                if t >= 2:
                    pl.semaphore_wait(credit_sem, 1)
                rdma = pltpu.make_async_remote_copy(
                    src_ref=acc.at[j],
                    dst_ref=recvb.at[slot],
                    send_sem=send_sems.at[slot],
                    recv_sem=recv_sems.at[slot],
                    device_id=(right,),
                    device_id_type=pl.DeviceIdType.MESH,
                )
                rdma.start()
                cp = pltpu.make_async_copy(
                    p_ref.at[pl.ds(q_recv * QR + j * BR, BR), :],
                    localb, local_sem,
                )
                cp.start()
                cp.wait()
                rdma.wait()
                total = (recvb[slot].astype(jnp.float32)
                         + localb[...].astype(jnp.float32))
                if s < N_DEV - 2:
                    acc[j, :, :] = total.astype(jnp.bfloat16)
                else:
                    val = jnp.maximum(total, 0.0)
                    outstage[...] = val
                    ocp = pltpu.make_async_copy(
                        outstage,
                        o_ref.at[pl.ds(d * QR + j * BR, BR), :],
                        out_sem,
                    )
                    ocp.start()
                    ocp.wait()
                    acc[j, :, :] = val.astype(jnp.bfloat16)
                if t <= T_TOTAL - 3:
                    pl.semaphore_signal(
                        credit_sem, inc=1,
                        device_id=(left,), device_id_type=pl.DeviceIdType.MESH,
                    )
                t += 1

        for s in range(N_DEV - 1):
            q_recv = lax.rem(d + 2 * N_DEV - 1 - s, N_DEV)
            for j in range(SUB):
                slot = t % 2
                if t >= 2:
                    pl.semaphore_wait(credit_sem, 1)
                rdma = pltpu.make_async_remote_copy(
                    src_ref=acc.at[j],
                    dst_ref=recvb.at[slot],
                    send_sem=send_sems.at[slot],
                    recv_sem=recv_sems.at[slot],
                    device_id=(right,),
                    device_id_type=pl.DeviceIdType.MESH,
                )
                rdma.start()
                rdma.wait()
                outstage[...] = recvb[slot].astype(jnp.float32)
                ocp = pltpu.make_async_copy(
                    outstage,
                    o_ref.at[pl.ds(q_recv * QR + j * BR, BR), :],
                    out_sem,
                )
                ocp.start()
                ocp.wait()
                if s < N_DEV - 2:
                    acc[j, :, :] = recvb[slot]
                if t <= T_TOTAL - 3:
                    pl.semaphore_signal(
                        credit_sem, inc=1,
                        device_id=(left,), device_id_type=pl.DeviceIdType.MESH,
                    )
                t += 1

    return pl.pallas_call(
        body,
        out_shape=jax.ShapeDtypeStruct((M, N), jnp.float32),
        in_specs=[pl.BlockSpec(memory_space=pltpu.ANY)],
        out_specs=pl.BlockSpec(memory_space=pltpu.ANY),
        scratch_shapes=[
            pltpu.VMEM((SUB, BR, N), jnp.bfloat16),
            pltpu.VMEM((2, BR, N), jnp.bfloat16),
            pltpu.VMEM((BR, N), jnp.bfloat16),
            pltpu.VMEM((BR, N), jnp.float32),
            pltpu.SemaphoreType.DMA((2,)),
            pltpu.SemaphoreType.DMA((2,)),
            pltpu.SemaphoreType.DMA,
            pltpu.SemaphoreType.DMA,
            pltpu.SemaphoreType.REGULAR,
        ],
        compiler_params=pltpu.CompilerParams(collective_id=0),
    )(p)


def kernel(x, w_mat):
    partial = jnp.dot(
        x, w_mat, preferred_element_type=jnp.float32
    ).astype(jnp.bfloat16)
    return _all_reduce_relu(partial)


# baseline (device time: 1469590 ns/iter reference)
import jax
import jax.numpy as jnp
from jax import lax
from jax.experimental import pallas as pl
from jax.experimental.pallas import tpu as pltpu

N_DEV = 4
M = 8192
N = 4096
QR = M // N_DEV
SUB = 4
BR = QR // SUB
T_TOTAL = 2 * (N_DEV - 1) * SUB


def _all_reduce_relu(p):

    def body(p_ref, o_ref, acc, recvb, localb, outstage,
             send_sems, recv_sems, local_sem, out_sem, credit_sem):
        d = lax.axis_index("i")
        right = lax.rem(d + 1, N_DEV)
        left = lax.rem(d + N_DEV - 1, N_DEV)

        barrier_sem = pltpu.get_barrier_semaphore()
        for nbr in (left, right):
            pl.semaphore_signal(
                barrier_sem, inc=1,
                device_id=(nbr,), device_id_type=pl.DeviceIdType.MESH,
            )
        pl.semaphore_wait(barrier_sem, 2)

        q0 = lax.rem(d + N_DEV - 1, N_DEV)
        for j in range(SUB):
            cp = pltpu.make_async_copy(
                p_ref.at[pl.ds(q0 * QR + j * BR, BR), :], acc.at[j], local_sem
            )
            cp.start()
            cp.wait()

        t = 0
        for s in range(N_DEV - 1):
            q_recv = lax.rem(d + 2 * N_DEV - 2 - s, N_DEV)
            for j in range(SUB):
                slot = t % 2
                if t >= 2:
                    pl.semaphore_wait(credit_sem, 1)
                rdma = pltpu.make_async_remote_copy(
                    src_ref=acc.at[j],
                    dst_ref=recvb.at[slot],
                    send_sem=send_sems.at[slot],
                    recv_sem=recv_sems.at[slot],
                    device_id=(right,),
                    device_id_type=pl.DeviceIdType.MESH,
                )
                rdma.start()
                cp = pltpu.make_async_copy(
                    p_ref.at[pl.ds(q_recv * QR + j * BR, BR), :],
                    localb, local_sem,
                )
                cp.start()
                cp.wait()
                rdma.wait()
                total = (recvb[slot].astype(jnp.float32)
                         + localb[...].astype(jnp.float32))
                if s < N_DEV - 2:
                    acc[j, :, :] = total.astype(jnp.bfloat16)
                else:
                    val = jnp.maximum(total, 0.0)
                    outstage[...] = val
                    ocp = pltpu.make_async_copy(
                        outstage,
                        o_ref.at[pl.ds(d * QR + j * BR, BR), :],
                        out_sem,
                    )
                    ocp.start()
                    ocp.wait()
                    acc[j, :, :] = val.astype(jnp.bfloat16)
                if t <= T_TOTAL - 3:
                    pl.semaphore_signal(
                        credit_sem, inc=1,
                        device_id=(left,), device_id_type=pl.DeviceIdType.MESH,
                    )
                t += 1

        for s in range(N_DEV - 1):
            q_recv = lax.rem(d + 2 * N_DEV - 1 - s, N_DEV)
            for j in range(SUB):
                slot = t % 2
                if t >= 2:
                    pl.semaphore_wait(credit_sem, 1)
                rdma = pltpu.make_async_remote_copy(
                    src_ref=acc.at[j],
                    dst_ref=recvb.at[slot],
                    send_sem=send_sems.at[slot],
                    recv_sem=recv_sems.at[slot],
                    device_id=(right,),
                    device_id_type=pl.DeviceIdType.MESH,
                )
                rdma.start()
                rdma.wait()
                outstage[...] = recvb[slot].astype(jnp.float32)
                ocp = pltpu.make_async_copy(
                    outstage,
                    o_ref.at[pl.ds(q_recv * QR + j * BR, BR), :],
                    out_sem,
                )
                ocp.start()
                ocp.wait()
                if s < N_DEV - 2:
                    acc[j, :, :] = recvb[slot]
                if t <= T_TOTAL - 3:
                    pl.semaphore_signal(
                        credit_sem, inc=1,
                        device_id=(left,), device_id_type=pl.DeviceIdType.MESH,
                    )
                t += 1

    return pl.pallas_call(
        body,
        out_shape=jax.ShapeDtypeStruct((M, N), jnp.float32),
        in_specs=[pl.BlockSpec(memory_space=pltpu.MemorySpace.HBM)],
        out_specs=pl.BlockSpec(memory_space=pltpu.MemorySpace.HBM),
        scratch_shapes=[
            pltpu.VMEM((SUB, BR, N), jnp.bfloat16),
            pltpu.VMEM((2, BR, N), jnp.bfloat16),
            pltpu.VMEM((BR, N), jnp.bfloat16),
            pltpu.VMEM((BR, N), jnp.float32),
            pltpu.SemaphoreType.DMA((2,)),
            pltpu.SemaphoreType.DMA((2,)),
            pltpu.SemaphoreType.DMA,
            pltpu.SemaphoreType.DMA,
            pltpu.SemaphoreType.REGULAR,
        ],
        compiler_params=pltpu.CompilerParams(
            collective_id=0, vmem_limit_bytes=60 * 1024 * 1024
        ),
    )(p)


def kernel(x, w_mat):
    partial = jnp.dot(
        x, w_mat, preferred_element_type=jnp.float32
    ).astype(jnp.bfloat16)
    return _all_reduce_relu(partial)


# device time: 943955 ns/iter; 1.5568x vs baseline; 1.5568x over previous
import jax
import jax.numpy as jnp
from jax import lax
from jax.experimental import pallas as pl
from jax.experimental.pallas import tpu as pltpu

N_DEV = 4
M = 8192
N = 4096
NC = N // 2
QR = M // N_DEV
SUB = 4
BR = QR // SUB
T_TOTAL = 2 * (N_DEV - 1) * SUB


def _all_reduce_relu(p):

    def body(p_ref, o_ref, acc, recvb, localb, outstage,
             send_sems, recv_sems, local_sems, out_sems, credit_sems):
        d = lax.axis_index("i")
        right = lax.rem(d + 1, N_DEV)
        left = lax.rem(d + N_DEV - 1, N_DEV)
        send_to = (right, left)
        recv_from = (left, right)

        barrier_sem = pltpu.get_barrier_semaphore()
        for nbr in (left, right):
            pl.semaphore_signal(
                barrier_sem, inc=1,
                device_id=(nbr,), device_id_type=pl.DeviceIdType.MESH,
            )
        pl.semaphore_wait(barrier_sem, 2)

        def send_quarter_rs(dir_, s):
            return lax.rem(d + (N_DEV - 1 - s if dir_ == 0 else 1 + s),
                           N_DEV)

        def recv_quarter_rs(dir_, s):
            return lax.rem(d + (2 * N_DEV - 2 - s if dir_ == 0 else 2 + s),
                           N_DEV)

        def recv_quarter_ag(dir_, s):
            return lax.rem(d + (2 * N_DEV - 1 - s if dir_ == 0 else 1 + s),
                           N_DEV)

        for dir_ in range(2):
            q0 = send_quarter_rs(dir_, 0)
            for j in range(SUB):
                cp = pltpu.make_async_copy(
                    p_ref.at[pl.ds(q0 * QR + j * BR, BR),
                             pl.ds(dir_ * NC, NC)],
                    acc.at[dir_, j], local_sems.at[dir_],
                )
                cp.start()
                cp.wait()

        def make_rdma(dir_, j, slot):
            return pltpu.make_async_remote_copy(
                src_ref=acc.at[dir_, j],
                dst_ref=recvb.at[dir_, slot],
                send_sem=send_sems.at[dir_, slot],
                recv_sem=recv_sems.at[dir_, slot],
                device_id=(send_to[dir_],),
                device_id_type=pl.DeviceIdType.MESH,
            )

        def signal_credit(dir_):
            pl.semaphore_signal(
                credit_sems.at[dir_], inc=1,
                device_id=(recv_from[dir_],),
                device_id_type=pl.DeviceIdType.MESH,
            )

        t = 0
        for s in range(N_DEV - 1):
            for j in range(SUB):
                slot = t % 2
                if t >= 2:
                    for dir_ in range(2):
                        pl.semaphore_wait(credit_sems.at[dir_], 1)
                rdmas = [make_rdma(dir_, j, slot) for dir_ in range(2)]
                for r in rdmas:
                    r.start()
                cps = []
                for dir_ in range(2):
                    q_recv = recv_quarter_rs(dir_, s)
                    cp = pltpu.make_async_copy(
                        p_ref.at[pl.ds(q_recv * QR + j * BR, BR),
                                 pl.ds(dir_ * NC, NC)],
                        localb.at[dir_], local_sems.at[dir_],
                    )
                    cp.start()
                    cps.append(cp)
                for cp in cps:
                    cp.wait()
                for r in rdmas:
                    r.wait()
                for dir_ in range(2):
                    total = (recvb[dir_, slot].astype(jnp.float32)
                             + localb[dir_].astype(jnp.float32))
                    if s < N_DEV - 2:
                        acc[dir_, j, :, :] = total.astype(jnp.bfloat16)
                    else:
                        val = jnp.maximum(total, 0.0)
                        outstage[dir_] = val
                        ocp = pltpu.make_async_copy(
                            outstage.at[dir_],
                            o_ref.at[pl.ds(d * QR + j * BR, BR),
                                     pl.ds(dir_ * NC, NC)],
                            out_sems.at[dir_],
                        )
                        ocp.start()
                        ocp.wait()
                        acc[dir_, j, :, :] = val.astype(jnp.bfloat16)
                if t <= T_TOTAL - 3:
                    for dir_ in range(2):
                        signal_credit(dir_)
                t += 1

        for s in range(N_DEV - 1):
            for j in range(SUB):
                slot = t % 2
                if t >= 2:
                    for dir_ in range(2):
                        pl.semaphore_wait(credit_sems.at[dir_], 1)
                rdmas = [make_rdma(dir_, j, slot) for dir_ in range(2)]
                for r in rdmas:
                    r.start()
                for r in rdmas:
                    r.wait()
                for dir_ in range(2):
                    q_recv = recv_quarter_ag(dir_, s)
                    outstage[dir_] = recvb[dir_, slot].astype(jnp.float32)
                    ocp = pltpu.make_async_copy(
                        outstage.at[dir_],
                        o_ref.at[pl.ds(q_recv * QR + j * BR, BR),
                                 pl.ds(dir_ * NC, NC)],
                        out_sems.at[dir_],
                    )
                    ocp.start()
                    ocp.wait()
                    if s < N_DEV - 2:
                        acc[dir_, j, :, :] = recvb[dir_, slot]
                if t <= T_TOTAL - 3:
                    for dir_ in range(2):
                        signal_credit(dir_)
                t += 1

    return pl.pallas_call(
        body,
        out_shape=jax.ShapeDtypeStruct((M, N), jnp.float32),
        in_specs=[pl.BlockSpec(memory_space=pltpu.MemorySpace.HBM)],
        out_specs=pl.BlockSpec(memory_space=pltpu.MemorySpace.HBM),
        scratch_shapes=[
            pltpu.VMEM((2, SUB, BR, NC), jnp.bfloat16),
            pltpu.VMEM((2, 2, BR, NC), jnp.bfloat16),
            pltpu.VMEM((2, BR, NC), jnp.bfloat16),
            pltpu.VMEM((2, BR, NC), jnp.float32),
            pltpu.SemaphoreType.DMA((2, 2)),
            pltpu.SemaphoreType.DMA((2, 2)),
            pltpu.SemaphoreType.DMA((2,)),
            pltpu.SemaphoreType.DMA((2,)),
            pltpu.SemaphoreType.REGULAR((2,)),
        ],
        compiler_params=pltpu.CompilerParams(
            collective_id=0, vmem_limit_bytes=60 * 1024 * 1024
        ),
    )(p)


def kernel(x, w_mat):
    partial = jnp.dot(
        x, w_mat, preferred_element_type=jnp.float32
    ).astype(jnp.bfloat16)
    return _all_reduce_relu(partial)


# device time: 826213 ns/iter; 1.7787x vs baseline; 1.1425x over previous
import jax
import jax.numpy as jnp
from jax import lax
from jax.experimental import pallas as pl
from jax.experimental.pallas import tpu as pltpu

N_DEV = 4
M = 8192
N = 4096
NC = N // 2
QR = M // N_DEV
SUB = 4
BR = QR // SUB
T_TOTAL = 2 * (N_DEV - 1) * SUB


def _all_reduce_relu(p):

    def body(p_ref, o_ref, acc, recvb, localb, outstage,
             send_sems, recv_sems, local_sems, out_sems, credit_sems):
        d = lax.axis_index("i")
        right = lax.rem(d + 1, N_DEV)
        left = lax.rem(d + N_DEV - 1, N_DEV)
        send_to = (right, left)
        recv_from = (left, right)

        barrier_sem = pltpu.get_barrier_semaphore()
        for nbr in (left, right):
            pl.semaphore_signal(
                barrier_sem, inc=1,
                device_id=(nbr,), device_id_type=pl.DeviceIdType.MESH,
            )
        pl.semaphore_wait(barrier_sem, 2)

        def recv_quarter_rs(dir_, s):
            return lax.rem(d + (2 * N_DEV - 2 - s if dir_ == 0 else 2 + s),
                           N_DEV)

        def recv_quarter_ag(dir_, s):
            return lax.rem(d + (2 * N_DEV - 1 - s if dir_ == 0 else 1 + s),
                           N_DEV)

        for dir_ in range(2):
            q0 = lax.rem(d + (N_DEV - 1 if dir_ == 0 else 1), N_DEV)
            for j in range(SUB):
                cp = pltpu.make_async_copy(
                    p_ref.at[pl.ds(q0 * QR + j * BR, BR),
                             pl.ds(dir_ * NC, NC)],
                    acc.at[dir_, j], local_sems.at[dir_, j % 2],
                )
                cp.start()
                cp.wait()

        def make_rdma(dir_, j, slot):
            return pltpu.make_async_remote_copy(
                src_ref=acc.at[dir_, j],
                dst_ref=recvb.at[dir_, slot],
                send_sem=send_sems.at[dir_, slot],
                recv_sem=recv_sems.at[dir_, slot],
                device_id=(send_to[dir_],),
                device_id_type=pl.DeviceIdType.MESH,
            )

        def signal_credit(dir_):
            pl.semaphore_signal(
                credit_sems.at[dir_], inc=1,
                device_id=(recv_from[dir_],),
                device_id_type=pl.DeviceIdType.MESH,
            )

        def consume(entry):
            phase, s, j, slot, t, rdmas, cps = entry
            for dir_ in range(2):
                rdmas[dir_].wait()
                if phase == "rs":
                    cps[dir_].wait()
                    total = (recvb[dir_, slot].astype(jnp.float32)
                             + localb[dir_, slot].astype(jnp.float32))
                    if s < N_DEV - 2:
                        acc[dir_, j, :, :] = total.astype(jnp.bfloat16)
                    else:
                        val = jnp.maximum(total, 0.0)
                        outstage[dir_] = val
                        ocp = pltpu.make_async_copy(
                            outstage.at[dir_],
                            o_ref.at[pl.ds(d * QR + j * BR, BR),
                                     pl.ds(dir_ * NC, NC)],
                            out_sems.at[dir_],
                        )
                        ocp.start()
                        ocp.wait()
                        acc[dir_, j, :, :] = val.astype(jnp.bfloat16)
                else:
                    q_recv = recv_quarter_ag(dir_, s)
                    outstage[dir_] = recvb[dir_, slot].astype(jnp.float32)
                    ocp = pltpu.make_async_copy(
                        outstage.at[dir_],
                        o_ref.at[pl.ds(q_recv * QR + j * BR, BR),
                                 pl.ds(dir_ * NC, NC)],
                        out_sems.at[dir_],
                    )
                    ocp.start()
                    ocp.wait()
                    if s < N_DEV - 2:
                        acc[dir_, j, :, :] = recvb[dir_, slot]
            if t <= T_TOTAL - 3:
                for dir_ in range(2):
                    signal_credit(dir_)

        schedule = [("rs", s, j) for s in range(N_DEV - 1)
                    for j in range(SUB)]
        schedule += [("ag", s, j) for s in range(N_DEV - 1)
                     for j in range(SUB)]

        pending = None
        for t, (phase, s, j) in enumerate(schedule):
            slot = t % 2
            if t >= 2:
                for dir_ in range(2):
                    pl.semaphore_wait(credit_sems.at[dir_], 1)
            rdmas = [make_rdma(dir_, j, slot) for dir_ in range(2)]
            for r in rdmas:
                r.start()
            cps = None
            if phase == "rs":
                cps = []
                for dir_ in range(2):
                    q_recv = recv_quarter_rs(dir_, s)
                    cp = pltpu.make_async_copy(
                        p_ref.at[pl.ds(q_recv * QR + j * BR, BR),
                                 pl.ds(dir_ * NC, NC)],
                        localb.at[dir_, slot], local_sems.at[dir_, slot],
                    )
                    cp.start()
                    cps.append(cp)
            if pending is not None:
                consume(pending)
            pending = (phase, s, j, slot, t, rdmas, cps)
        consume(pending)

    return pl.pallas_call(
        body,
        out_shape=jax.ShapeDtypeStruct((M, N), jnp.float32),
        in_specs=[pl.BlockSpec(memory_space=pltpu.MemorySpace.HBM)],
        out_specs=pl.BlockSpec(memory_space=pltpu.MemorySpace.HBM),
        scratch_shapes=[
            pltpu.VMEM((2, SUB, BR, NC), jnp.bfloat16),
            pltpu.VMEM((2, 2, BR, NC), jnp.bfloat16),
            pltpu.VMEM((2, 2, BR, NC), jnp.bfloat16),
            pltpu.VMEM((2, BR, NC), jnp.float32),
            pltpu.SemaphoreType.DMA((2, 2)),
            pltpu.SemaphoreType.DMA((2, 2)),
            pltpu.SemaphoreType.DMA((2, 2)),
            pltpu.SemaphoreType.DMA((2,)),
            pltpu.SemaphoreType.REGULAR((2,)),
        ],
        compiler_params=pltpu.CompilerParams(
            collective_id=0, vmem_limit_bytes=60 * 1024 * 1024
        ),
    )(p)


def kernel(x, w_mat):
    partial = jnp.dot(
        x, w_mat, preferred_element_type=jnp.float32
    ).astype(jnp.bfloat16)
    return _all_reduce_relu(partial)


# device time: 703215 ns/iter; 2.0898x vs baseline; 1.1749x over previous
import jax
import jax.numpy as jnp
from jax import lax
from jax.experimental import pallas as pl
from jax.experimental.pallas import tpu as pltpu

N_DEV = 4
M = 8192
K = 2048
N = 4096
NC = N // 2
QR = M // N_DEV
SUB = 4
BR = QR // SUB
RS_T = (N_DEV - 1) * SUB
T_TOTAL = 2 * RS_T


def _fused(x, w):
    def body(x_ref, w_ref, o_ref, wv, acc, recvb, cx, px, outstage,
             send_sems, recv_sems, cx_sems, px_sems, w_sem, out_sem,
             credit_sems):
        d = lax.axis_index("i")
        right = lax.rem(d + 1, N_DEV)
        left = lax.rem(d + N_DEV - 1, N_DEV)
        send_to = (right, left)
        recv_from = (left, right)

        barrier_sem = pltpu.get_barrier_semaphore()
        for nbr in (left, right):
            pl.semaphore_signal(
                barrier_sem, inc=1,
                device_id=(nbr,), device_id_type=pl.DeviceIdType.MESH,
            )
        pl.semaphore_wait(barrier_sem, 2)

        for dir_ in range(2):
            cp = pltpu.make_async_copy(
                w_ref.at[:, pl.ds(dir_ * NC, NC)], wv.at[dir_], w_sem
            )
            cp.start()
            cp.wait()

        def recv_quarter_rs(dir_, s):
            return lax.rem(d + (2 * N_DEV - 2 - s if dir_ == 0
                                else 2 + s), N_DEV)

        def recv_quarter_ag(dir_, s):
            return lax.rem(d + (2 * N_DEV - 1 - s if dir_ == 0
                                else 1 + s), N_DEV)

        prod_q = (lax.rem(d + N_DEV - 1, N_DEV), lax.rem(d + 1, N_DEV))

        def px_copy(dir_, j):
            return pltpu.make_async_copy(
                x_ref.at[pl.ds(prod_q[dir_] * QR + j * BR, BR), :],
                px.at[dir_], px_sems.at[dir_],
            )

        def cx_copy(dir_, s, j, slot):
            return pltpu.make_async_copy(
                x_ref.at[pl.ds(recv_quarter_rs(dir_, s) * QR + j * BR,
                               BR), :],
                cx.at[dir_, slot], cx_sems.at[dir_, slot],
            )

        def produce(j):
            for dir_ in range(2):
                px_copy(dir_, j).wait()
                acc[dir_, j] = jnp.dot(
                    px[dir_], wv[dir_],
                    preferred_element_type=jnp.float32,
                ).astype(jnp.bfloat16)

        def make_rdma(dir_, j, slot):
            return pltpu.make_async_remote_copy(
                src_ref=acc.at[dir_, j],
                dst_ref=recvb.at[dir_, slot],
                send_sem=send_sems.at[dir_, slot],
                recv_sem=recv_sems.at[dir_, slot],
                device_id=(send_to[dir_],),
                device_id_type=pl.DeviceIdType.MESH,
            )

        def signal_credit(dir_):
            pl.semaphore_signal(
                credit_sems.at[dir_], inc=1,
                device_id=(recv_from[dir_],),
                device_id_type=pl.DeviceIdType.MESH,
            )

        def store_out(dir_, row_start, value):
            outstage[...] = value
            ocp = pltpu.make_async_copy(
                outstage,
                o_ref.at[pl.ds(row_start, BR), pl.ds(dir_ * NC, NC)],
                out_sem,
            )
            ocp.start()
            ocp.wait()

        def sched(idx):
            in_rs = idx < RS_T
            s = jnp.where(in_rs, idx // SUB, (idx - RS_T) // SUB)
            return in_rs, s, lax.rem(idx, SUB), lax.rem(idx, 2)

        def consume(c):
            in_rs, s, j, slot = sched(c)
            for dir_ in range(2):
                make_rdma(dir_, j, slot).wait()

                @pl.when(in_rs)
                def _():
                    cx_copy(dir_, s, j, slot).wait()
                    total = jnp.dot(
                        cx[dir_, slot], wv[dir_],
                        preferred_element_type=jnp.float32,
                    ) + recvb[dir_, slot].astype(jnp.float32)

                    @pl.when(s < N_DEV - 2)
                    def _():
                        acc[dir_, j] = total.astype(jnp.bfloat16)

                    @pl.when(s == N_DEV - 2)
                    def _():
                        val = jnp.maximum(total, 0.0)
                        store_out(dir_, d * QR + j * BR, val)
                        acc[dir_, j] = val.astype(jnp.bfloat16)

                @pl.when(jnp.logical_not(in_rs))
                def _():
                    store_out(dir_, recv_quarter_ag(dir_, s) * QR + j * BR,
                              recvb[dir_, slot].astype(jnp.float32))

                    @pl.when(s < N_DEV - 2)
                    def _():
                        acc[dir_, j] = recvb[dir_, slot]

            @pl.when(c <= T_TOTAL - 3)
            def _():
                for dir_ in range(2):
                    signal_credit(dir_)

        for dir_ in range(2):
            px_copy(dir_, 0).start()
        produce(0)
        for dir_ in range(2):
            px_copy(dir_, 1).start()

        def loop_body(i, carry):
            in_rs, s, j, slot = sched(i)

            @pl.when(i >= 2)
            def _():
                for dir_ in range(2):
                    pl.semaphore_wait(credit_sems.at[dir_], 1)

            for dir_ in range(2):
                make_rdma(dir_, j, slot).start()

            @pl.when(in_rs)
            def _():
                for dir_ in range(2):
                    cx_copy(dir_, s, j, slot).start()

            @pl.when(i < SUB - 1)
            def _():
                produce(i + 1)

                @pl.when(i < SUB - 2)
                def _():
                    for dir_ in range(2):
                        px_copy(dir_, i + 2).start()

            @pl.when(i >= 1)
            def _():
                consume(i - 1)

            return carry

        lax.fori_loop(0, T_TOTAL, loop_body, 0)
        consume(T_TOTAL - 1)

    return pl.pallas_call(
        body,
        out_shape=jax.ShapeDtypeStruct((M, N), jnp.float32),
        in_specs=[pl.BlockSpec(memory_space=pltpu.MemorySpace.HBM),
                  pl.BlockSpec(memory_space=pltpu.MemorySpace.HBM)],
        out_specs=pl.BlockSpec(memory_space=pltpu.MemorySpace.HBM),
        scratch_shapes=[
            pltpu.VMEM((2, K, NC), jnp.bfloat16),
            pltpu.VMEM((2, SUB, BR, NC), jnp.bfloat16),
            pltpu.VMEM((2, 2, BR, NC), jnp.bfloat16),
            pltpu.VMEM((2, 2, BR, K), jnp.bfloat16),
            pltpu.VMEM((2, BR, K), jnp.bfloat16),
            pltpu.VMEM((BR, NC), jnp.float32),
            pltpu.SemaphoreType.DMA((2, 2)),
            pltpu.SemaphoreType.DMA((2, 2)),
            pltpu.SemaphoreType.DMA((2, 2)),
            pltpu.SemaphoreType.DMA((2,)),
            pltpu.SemaphoreType.DMA,
            pltpu.SemaphoreType.DMA,
            pltpu.SemaphoreType.REGULAR((2,)),
        ],
        compiler_params=pltpu.CompilerParams(
            collective_id=0, vmem_limit_bytes=64 * 1024 * 1024
        ),
    )(x, w)


def kernel(x, w_mat):
    return _fused(x.astype(jnp.bfloat16), w_mat.astype(jnp.bfloat16))


# device time: 702890 ns/iter; 2.0908x vs baseline; 1.0005x over previous
import jax
import jax.numpy as jnp
from jax import lax
from jax.experimental import pallas as pl
from jax.experimental.pallas import tpu as pltpu

N_DEV = 4
M = 8192
K = 2048
N = 4096
NC = N // 2
QR = M // N_DEV
SUB = 4
BR = QR // SUB
RS_T = (N_DEV - 1) * SUB
T_TOTAL = 2 * RS_T


def _fused(x, w):
    def body(x_ref, w_ref, o_ref, wv, acc, recvb, cx, px, outstage,
             send_sems, recv_sems, cx_sems, px_sems, w_sem, out_sem,
             credit_sems):
        d = lax.axis_index("i")
        right = lax.rem(d + 1, N_DEV)
        left = lax.rem(d + N_DEV - 1, N_DEV)
        send_to = (right, left)
        recv_from = (left, right)

        prod_q = (lax.rem(d + N_DEV - 1, N_DEV), lax.rem(d + 1, N_DEV))

        def px_copy(dir_, j):
            return pltpu.make_async_copy(
                x_ref.at[pl.ds(prod_q[dir_] * QR + j * BR, BR), :],
                px.at[dir_], px_sems.at[dir_],
            )

        def produce(j):
            for dir_ in range(2):
                px_copy(dir_, j).wait()
                acc[dir_, j] = jnp.dot(
                    px[dir_], wv[dir_],
                    preferred_element_type=jnp.float32,
                ).astype(jnp.bfloat16)

        for dir_ in range(2):
            cp = pltpu.make_async_copy(
                w_ref.at[:, pl.ds(dir_ * NC, NC)], wv.at[dir_], w_sem
            )
            cp.start()
            cp.wait()
        for dir_ in range(2):
            px_copy(dir_, 0).start()
        produce(0)
        for dir_ in range(2):
            px_copy(dir_, 1).start()

        barrier_sem = pltpu.get_barrier_semaphore()
        for nbr in (left, right):
            pl.semaphore_signal(
                barrier_sem, inc=1,
                device_id=(nbr,), device_id_type=pl.DeviceIdType.MESH,
            )
        pl.semaphore_wait(barrier_sem, 2)

        def recv_quarter_rs(dir_, s):
            return lax.rem(d + (2 * N_DEV - 2 - s if dir_ == 0
                                else 2 + s), N_DEV)

        def recv_quarter_ag(dir_, s):
            return lax.rem(d + (2 * N_DEV - 1 - s if dir_ == 0
                                else 1 + s), N_DEV)

        def cx_copy(dir_, s, j, slot):
            return pltpu.make_async_copy(
                x_ref.at[pl.ds(recv_quarter_rs(dir_, s) * QR + j * BR,
                               BR), :],
                cx.at[dir_, slot], cx_sems.at[dir_, slot],
            )

        def make_rdma(dir_, j, slot):
            return pltpu.make_async_remote_copy(
                src_ref=acc.at[dir_, j],
                dst_ref=recvb.at[dir_, slot],
                send_sem=send_sems.at[dir_, slot],
                recv_sem=recv_sems.at[dir_, slot],
                device_id=(send_to[dir_],),
                device_id_type=pl.DeviceIdType.MESH,
            )

        def signal_credit(dir_):
            pl.semaphore_signal(
                credit_sems.at[dir_], inc=1,
                device_id=(recv_from[dir_],),
                device_id_type=pl.DeviceIdType.MESH,
            )

        def store_out(dir_, row_start, value):
            outstage[...] = value
            ocp = pltpu.make_async_copy(
                outstage,
                o_ref.at[pl.ds(row_start, BR), pl.ds(dir_ * NC, NC)],
                out_sem,
            )
            ocp.start()
            ocp.wait()

        def sched(idx):
            in_rs = idx < RS_T
            s = jnp.where(in_rs, idx // SUB, (idx - RS_T) // SUB)
            return in_rs, s, lax.rem(idx, SUB), lax.rem(idx, 2)

        def consume(c):
            in_rs, s, j, slot = sched(c)
            for dir_ in range(2):
                make_rdma(dir_, j, slot).wait()

                @pl.when(in_rs)
                def _():
                    cx_copy(dir_, s, j, slot).wait()
                    total = jnp.dot(
                        cx[dir_, slot], wv[dir_],
                        preferred_element_type=jnp.float32,
                    ) + recvb[dir_, slot].astype(jnp.float32)

                    @pl.when(s < N_DEV - 2)
                    def _():
                        acc[dir_, j] = total.astype(jnp.bfloat16)

                    @pl.when(s == N_DEV - 2)
                    def _():
                        val = jnp.maximum(total, 0.0)
                        store_out(dir_, d * QR + j * BR, val)
                        acc[dir_, j] = val.astype(jnp.bfloat16)

                @pl.when(jnp.logical_not(in_rs))
                def _():
                    store_out(dir_, recv_quarter_ag(dir_, s) * QR + j * BR,
                              recvb[dir_, slot].astype(jnp.float32))

                    @pl.when(s < N_DEV - 2)
                    def _():
                        acc[dir_, j] = recvb[dir_, slot]

            @pl.when(c <= T_TOTAL - 3)
            def _():
                for dir_ in range(2):
                    signal_credit(dir_)

        def loop_body(i, carry):
            in_rs, s, j, slot = sched(i)

            @pl.when(i >= 2)
            def _():
                for dir_ in range(2):
                    pl.semaphore_wait(credit_sems.at[dir_], 1)

            for dir_ in range(2):
                make_rdma(dir_, j, slot).start()

            @pl.when(in_rs)
            def _():
                for dir_ in range(2):
                    cx_copy(dir_, s, j, slot).start()

            @pl.when(i < SUB - 1)
            def _():
                produce(i + 1)

                @pl.when(i < SUB - 2)
                def _():
                    for dir_ in range(2):
                        px_copy(dir_, i + 2).start()

            @pl.when(i >= 1)
            def _():
                consume(i - 1)

            return carry

        lax.fori_loop(0, T_TOTAL, loop_body, 0)
        consume(T_TOTAL - 1)

    return pl.pallas_call(
        body,
        out_shape=jax.ShapeDtypeStruct((M, N), jnp.float32),
        in_specs=[pl.BlockSpec(memory_space=pltpu.MemorySpace.HBM),
                  pl.BlockSpec(memory_space=pltpu.MemorySpace.HBM)],
        out_specs=pl.BlockSpec(memory_space=pltpu.MemorySpace.HBM),
        scratch_shapes=[
            pltpu.VMEM((2, K, NC), jnp.bfloat16),
            pltpu.VMEM((2, SUB, BR, NC), jnp.bfloat16),
            pltpu.VMEM((2, 2, BR, NC), jnp.bfloat16),
            pltpu.VMEM((2, 2, BR, K), jnp.bfloat16),
            pltpu.VMEM((2, BR, K), jnp.bfloat16),
            pltpu.VMEM((BR, NC), jnp.float32),
            pltpu.SemaphoreType.DMA((2, 2)),
            pltpu.SemaphoreType.DMA((2, 2)),
            pltpu.SemaphoreType.DMA((2, 2)),
            pltpu.SemaphoreType.DMA((2,)),
            pltpu.SemaphoreType.DMA,
            pltpu.SemaphoreType.DMA,
            pltpu.SemaphoreType.REGULAR((2,)),
        ],
        compiler_params=pltpu.CompilerParams(
            collective_id=0, vmem_limit_bytes=64 * 1024 * 1024
        ),
    )(x, w)


def kernel(x, w_mat):
    return _fused(x.astype(jnp.bfloat16), w_mat.astype(jnp.bfloat16))


# device time: 658064 ns/iter; 2.2332x vs baseline; 1.0681x over previous
import jax
import jax.numpy as jnp
from jax import lax
from jax.experimental import pallas as pl
from jax.experimental.pallas import tpu as pltpu

N_DEV = 4
M = 8192
K = 2048
N = 4096
NC = N // 2
QR = M // N_DEV
SUB = 4
BR = QR // SUB
RS_T = (N_DEV - 1) * SUB
T_TOTAL = 2 * RS_T


def _fused(x, w):
    def body(x_ref, w_ref, o_ref, wv, acc, recvb, cx, px,
             send_sems, recv_sems, cx_sems, px_sems, w_sem, out_sem,
             credit_sems):
        d = lax.axis_index("i")
        right = lax.rem(d + 1, N_DEV)
        left = lax.rem(d + N_DEV - 1, N_DEV)
        send_to = (right, left)
        recv_from = (left, right)

        prod_q = (lax.rem(d + N_DEV - 1, N_DEV), lax.rem(d + 1, N_DEV))

        def px_copy(dir_, j):
            return pltpu.make_async_copy(
                x_ref.at[pl.ds(prod_q[dir_] * QR + j * BR, BR), :],
                px.at[dir_], px_sems.at[dir_],
            )

        def produce(j):
            for dir_ in range(2):
                px_copy(dir_, j).wait()
                acc[dir_, j] = jnp.dot(
                    px[dir_], wv[dir_],
                    preferred_element_type=jnp.float32,
                ).astype(jnp.bfloat16)

        for dir_ in range(2):
            cp = pltpu.make_async_copy(
                w_ref.at[:, pl.ds(dir_ * NC, NC)], wv.at[dir_], w_sem
            )
            cp.start()
            cp.wait()
        for dir_ in range(2):
            px_copy(dir_, 0).start()
        produce(0)
        for dir_ in range(2):
            px_copy(dir_, 1).start()

        barrier_sem = pltpu.get_barrier_semaphore()
        for nbr in (left, right):
            pl.semaphore_signal(
                barrier_sem, inc=1,
                device_id=(nbr,), device_id_type=pl.DeviceIdType.MESH,
            )
        pl.semaphore_wait(barrier_sem, 2)

        def recv_quarter_rs(dir_, s):
            return lax.rem(d + (2 * N_DEV - 2 - s if dir_ == 0
                                else 2 + s), N_DEV)

        def recv_quarter_ag(dir_, s):
            return lax.rem(d + (2 * N_DEV - 1 - s if dir_ == 0
                                else 1 + s), N_DEV)

        def cx_copy(dir_, s, j, slot):
            return pltpu.make_async_copy(
                x_ref.at[pl.ds(recv_quarter_rs(dir_, s) * QR + j * BR,
                               BR), :],
                cx.at[dir_, slot], cx_sems.at[dir_, slot],
            )

        def make_rdma(dir_, j, slot):
            return pltpu.make_async_remote_copy(
                src_ref=acc.at[dir_, j],
                dst_ref=recvb.at[dir_, slot],
                send_sem=send_sems.at[dir_, slot],
                recv_sem=recv_sems.at[dir_, slot],
                device_id=(send_to[dir_],),
                device_id_type=pl.DeviceIdType.MESH,
            )

        def signal_credit(dir_):
            pl.semaphore_signal(
                credit_sems.at[dir_], inc=1,
                device_id=(recv_from[dir_],),
                device_id_type=pl.DeviceIdType.MESH,
            )

        def store_out(dir_, row_start, src):
            ocp = pltpu.make_async_copy(
                src,
                o_ref.at[pl.ds(row_start, BR), pl.ds(dir_ * NC, NC)],
                out_sem,
            )
            ocp.start()
            ocp.wait()

        def sched(idx):
            in_rs = idx < RS_T
            s = jnp.where(in_rs, idx // SUB, (idx - RS_T) // SUB)
            return in_rs, s, lax.rem(idx, SUB), lax.rem(idx, 2)

        def consume(c):
            in_rs, s, j, slot = sched(c)
            for dir_ in range(2):
                make_rdma(dir_, j, slot).wait()

                @pl.when(in_rs)
                def _():
                    cx_copy(dir_, s, j, slot).wait()
                    total = jnp.dot(
                        cx[dir_, slot], wv[dir_],
                        preferred_element_type=jnp.float32,
                    ) + recvb[dir_, slot].astype(jnp.float32)

                    @pl.when(s < N_DEV - 2)
                    def _():
                        acc[dir_, j] = total.astype(jnp.bfloat16)

                    @pl.when(s == N_DEV - 2)
                    def _():
                        acc[dir_, j] = jnp.maximum(total, 0.0).astype(
                            jnp.bfloat16)
                        store_out(dir_, d * QR + j * BR, acc.at[dir_, j])

                @pl.when(jnp.logical_not(in_rs))
                def _():
                    store_out(dir_, recv_quarter_ag(dir_, s) * QR + j * BR,
                              recvb.at[dir_, slot])

                    @pl.when(s < N_DEV - 2)
                    def _():
                        acc[dir_, j] = recvb[dir_, slot]

            @pl.when(c <= T_TOTAL - 3)
            def _():
                for dir_ in range(2):
                    signal_credit(dir_)

        def loop_body(i, carry):
            in_rs, s, j, slot = sched(i)

            @pl.when(i >= 2)
            def _():
                for dir_ in range(2):
                    pl.semaphore_wait(credit_sems.at[dir_], 1)

            for dir_ in range(2):
                make_rdma(dir_, j, slot).start()

            @pl.when(in_rs)
            def _():
                for dir_ in range(2):
                    cx_copy(dir_, s, j, slot).start()

            @pl.when(i < SUB - 1)
            def _():
                produce(i + 1)

                @pl.when(i < SUB - 2)
                def _():
                    for dir_ in range(2):
                        px_copy(dir_, i + 2).start()

            @pl.when(i >= 1)
            def _():
                consume(i - 1)

            return carry

        lax.fori_loop(0, T_TOTAL, loop_body, 0)
        consume(T_TOTAL - 1)

    return pl.pallas_call(
        body,
        out_shape=jax.ShapeDtypeStruct((M, N), jnp.bfloat16),
        in_specs=[pl.BlockSpec(memory_space=pltpu.MemorySpace.HBM),
                  pl.BlockSpec(memory_space=pltpu.MemorySpace.HBM)],
        out_specs=pl.BlockSpec(memory_space=pltpu.MemorySpace.HBM),
        scratch_shapes=[
            pltpu.VMEM((2, K, NC), jnp.bfloat16),
            pltpu.VMEM((2, SUB, BR, NC), jnp.bfloat16),
            pltpu.VMEM((2, 2, BR, NC), jnp.bfloat16),
            pltpu.VMEM((2, 2, BR, K), jnp.bfloat16),
            pltpu.VMEM((2, BR, K), jnp.bfloat16),
            pltpu.SemaphoreType.DMA((2, 2)),
            pltpu.SemaphoreType.DMA((2, 2)),
            pltpu.SemaphoreType.DMA((2, 2)),
            pltpu.SemaphoreType.DMA((2,)),
            pltpu.SemaphoreType.DMA,
            pltpu.SemaphoreType.DMA,
            pltpu.SemaphoreType.REGULAR((2,)),
        ],
        compiler_params=pltpu.CompilerParams(
            collective_id=0, vmem_limit_bytes=64 * 1024 * 1024
        ),
    )(x, w)


def kernel(x, w_mat):
    return _fused(x.astype(jnp.bfloat16), w_mat.astype(jnp.bfloat16))


# device time: 657335 ns/iter; 2.2357x vs baseline; 1.0011x over previous
import jax
import jax.numpy as jnp
from jax import lax
from jax.experimental import pallas as pl
from jax.experimental.pallas import tpu as pltpu

N_DEV = 4
M = 8192
K = 2048
N = 4096
NC = N // 2
QR = M // N_DEV
SUB = 4
BR = QR // SUB
RS_T = (N_DEV - 1) * SUB
T_TOTAL = 2 * RS_T


def _fused(x, w):
    def body(x_ref, w_ref, o_ref, wv, acc, recvb, cx, px,
             send_sems, recv_sems, cx_sems, px_sems, w_sem, out_sem,
             credit_sems):
        d = lax.axis_index("i")
        right = lax.rem(d + 1, N_DEV)
        left = lax.rem(d + N_DEV - 1, N_DEV)
        send_to = (right, left)
        recv_from = (left, right)

        prod_q = (lax.rem(d + N_DEV - 1, N_DEV), lax.rem(d + 1, N_DEV))

        def px_copy(dir_, j):
            return pltpu.make_async_copy(
                x_ref.at[pl.ds(prod_q[dir_] * QR + j * BR, BR), :],
                px.at[dir_], px_sems.at[dir_],
            )

        def produce(j):
            for dir_ in range(2):
                px_copy(dir_, j).wait()
                acc[dir_, j] = jnp.dot(
                    px[dir_], wv[:, dir_ * NC:(dir_ + 1) * NC],
                    preferred_element_type=jnp.float32,
                ).astype(jnp.bfloat16)

        cp = pltpu.make_async_copy(w_ref, wv, w_sem)
        cp.start()
        cp.wait()
        for dir_ in range(2):
            px_copy(dir_, 0).start()
        produce(0)
        for dir_ in range(2):
            px_copy(dir_, 1).start()

        barrier_sem = pltpu.get_barrier_semaphore()
        for nbr in (left, right):
            pl.semaphore_signal(
                barrier_sem, inc=1,
                device_id=(nbr,), device_id_type=pl.DeviceIdType.MESH,
            )
        pl.semaphore_wait(barrier_sem, 2)

        def recv_quarter_rs(dir_, s):
            return lax.rem(d + (2 * N_DEV - 2 - s if dir_ == 0
                                else 2 + s), N_DEV)

        def recv_quarter_ag(dir_, s):
            return lax.rem(d + (2 * N_DEV - 1 - s if dir_ == 0
                                else 1 + s), N_DEV)

        def cx_copy(dir_, s, j, slot):
            return pltpu.make_async_copy(
                x_ref.at[pl.ds(recv_quarter_rs(dir_, s) * QR + j * BR,
                               BR), :],
                cx.at[dir_, slot], cx_sems.at[dir_, slot],
            )

        def make_rdma(dir_, j, slot):
            return pltpu.make_async_remote_copy(
                src_ref=acc.at[dir_, j],
                dst_ref=recvb.at[dir_, slot],
                send_sem=send_sems.at[dir_, slot],
                recv_sem=recv_sems.at[dir_, slot],
                device_id=(send_to[dir_],),
                device_id_type=pl.DeviceIdType.MESH,
            )

        def signal_credit(dir_):
            pl.semaphore_signal(
                credit_sems.at[dir_], inc=1,
                device_id=(recv_from[dir_],),
                device_id_type=pl.DeviceIdType.MESH,
            )

        def store_out(dir_, row_start, src):
            ocp = pltpu.make_async_copy(
                src,
                o_ref.at[pl.ds(row_start, BR), pl.ds(dir_ * NC, NC)],
                out_sem,
            )
            ocp.start()
            ocp.wait()

        def sched(idx):
            in_rs = idx < RS_T
            s = jnp.where(in_rs, idx // SUB, (idx - RS_T) // SUB)
            return in_rs, s, lax.rem(idx, SUB), lax.rem(idx, 2)

        def consume(c):
            in_rs, s, j, slot = sched(c)
            for dir_ in range(2):
                make_rdma(dir_, j, slot).wait()

            @pl.when(in_rs)
            def _():
                @pl.when(s == 1)
                def _():
                    for dir_ in range(2):
                        cx_copy(dir_, s, j, slot).wait()
                        total = jnp.dot(
                            cx[dir_, slot],
                            wv[:, dir_ * NC:(dir_ + 1) * NC],
                            preferred_element_type=jnp.float32,
                        ) + recvb[dir_, slot].astype(jnp.float32)
                        acc[dir_, j] = total.astype(jnp.bfloat16)

                @pl.when(s != 1)
                def _():
                    cx_copy(0, s, j, slot).wait()
                    full = jnp.dot(
                        cx[0, slot], wv[...],
                        preferred_element_type=jnp.float32,
                    )
                    for dir_ in range(2):
                        total = (full[:, dir_ * NC:(dir_ + 1) * NC]
                                 + recvb[dir_, slot].astype(jnp.float32))

                        @pl.when(s == 0)
                        def _():
                            acc[dir_, j] = total.astype(jnp.bfloat16)

                        @pl.when(s == N_DEV - 2)
                        def _():
                            acc[dir_, j] = jnp.maximum(total, 0.0).astype(
                                jnp.bfloat16)
                            store_out(dir_, d * QR + j * BR,
                                      acc.at[dir_, j])

            @pl.when(jnp.logical_not(in_rs))
            def _():
                for dir_ in range(2):
                    store_out(dir_, recv_quarter_ag(dir_, s) * QR + j * BR,
                              recvb.at[dir_, slot])

                    @pl.when(s < N_DEV - 2)
                    def _():
                        acc[dir_, j] = recvb[dir_, slot]

            @pl.when(c <= T_TOTAL - 3)
            def _():
                for dir_ in range(2):
                    signal_credit(dir_)

        def loop_body(i, carry):
            in_rs, s, j, slot = sched(i)

            @pl.when(i >= 2)
            def _():
                for dir_ in range(2):
                    pl.semaphore_wait(credit_sems.at[dir_], 1)

            for dir_ in range(2):
                make_rdma(dir_, j, slot).start()

            @pl.when(in_rs)
            def _():
                cx_copy(0, s, j, slot).start()

                @pl.when(s == 1)
                def _():
                    cx_copy(1, s, j, slot).start()

            @pl.when(i < SUB - 1)
            def _():
                produce(i + 1)

                @pl.when(i < SUB - 2)
                def _():
                    for dir_ in range(2):
                        px_copy(dir_, i + 2).start()

            @pl.when(i >= 1)
            def _():
                consume(i - 1)

            return carry

        lax.fori_loop(0, T_TOTAL, loop_body, 0)
        consume(T_TOTAL - 1)

    return pl.pallas_call(
        body,
        out_shape=jax.ShapeDtypeStruct((M, N), jnp.bfloat16),
        in_specs=[pl.BlockSpec(memory_space=pltpu.MemorySpace.HBM),
                  pl.BlockSpec(memory_space=pltpu.MemorySpace.HBM)],
        out_specs=pl.BlockSpec(memory_space=pltpu.MemorySpace.HBM),
        scratch_shapes=[
            pltpu.VMEM((K, N), jnp.bfloat16),
            pltpu.VMEM((2, SUB, BR, NC), jnp.bfloat16),
            pltpu.VMEM((2, 2, BR, NC), jnp.bfloat16),
            pltpu.VMEM((2, 2, BR, K), jnp.bfloat16),
            pltpu.VMEM((2, BR, K), jnp.bfloat16),
            pltpu.SemaphoreType.DMA((2, 2)),
            pltpu.SemaphoreType.DMA((2, 2)),
            pltpu.SemaphoreType.DMA((2, 2)),
            pltpu.SemaphoreType.DMA((2,)),
            pltpu.SemaphoreType.DMA,
            pltpu.SemaphoreType.DMA,
            pltpu.SemaphoreType.REGULAR((2,)),
        ],
        compiler_params=pltpu.CompilerParams(
            collective_id=0, vmem_limit_bytes=64 * 1024 * 1024
        ),
    )(x, w)


def kernel(x, w_mat):
    return _fused(x.astype(jnp.bfloat16), w_mat.astype(jnp.bfloat16))
